# baseline (speedup 1.0000x reference)
import functools

import jax
import jax.numpy as jnp
from jax import lax
from jax.experimental import pallas as pl
from jax.experimental.pallas import tpu as pltpu
from jax.experimental.pallas import tpu_sc as plsc

D_MODEL = 1024
SCALE = 32.0

NUM_CORES = 2
NUM_SUBCORES = 16
LANES = 16
NW = NUM_CORES * NUM_SUBCORES

CHUNK = 32
NBUF = 3


@functools.partial(jax.jit, static_argnames=("total_b",))
def _embed(x_flat, table, total_b):
    b_per_w = total_b // NW
    n_chunks = b_per_w // CHUNK
    n_groups = (n_chunks - 2) // NBUF
    mesh = plsc.VectorSubcoreMesh(core_axis_name="c", subcore_axis_name="s")

    @functools.partial(
        pl.kernel,
        out_type=jax.ShapeDtypeStruct((total_b, D_MODEL), jnp.float32),
        mesh=mesh,
        scratch_types=[
            pltpu.VMEM((b_per_w,), jnp.int32),
            [pltpu.VMEM((CHUNK, D_MODEL), jnp.float32) for _ in range(NBUF)],
            [pltpu.SemaphoreType.DMA for _ in range(NBUF)],
            [pltpu.SemaphoreType.DMA for _ in range(NBUF)],
        ],
    )
    def k(x_hbm, table_hbm, out_hbm, idx_v, rows, gsems, wsems):
        wid = lax.axis_index("s") * NUM_CORES + lax.axis_index("c")
        base = wid * b_per_w
        pltpu.sync_copy(x_hbm.at[pl.ds(base, b_per_w)], idx_v)

        def gather_desc(c, b):
            return pltpu.make_async_copy(
                table_hbm.at[idx_v.at[pl.ds(c * CHUNK, CHUNK)]], rows[b], gsems[b]
            )

        def write_desc(c, b):
            return pltpu.make_async_copy(
                rows[b], out_hbm.at[pl.ds(base + c * CHUNK, CHUNK)], wsems[b]
            )

        def scale_buf(b):
            @plsc.parallel_loop(0, CHUNK)
            def scale_row(r):
                for j in range(D_MODEL // LANES):
                    v = rows[b][r, pl.ds(j * LANES, LANES)]
                    rows[b][r, pl.ds(j * LANES, LANES)] = v * SCALE

        gather_desc(0, 0).start()
        gather_desc(1, 1).start()

        def group_body(g, _):
            for b in range(NBUF):
                c = g * NBUF + b
                bp = (b + 2) % NBUF

                @pl.when(c >= 1)
                def _drain_write():
                    write_desc(c - 1, bp).wait()

                gather_desc(c + 2, bp).start()
                gather_desc(c, b).wait()
                scale_buf(b)
                write_desc(c, b).start()
            return 0

        lax.fori_loop(0, n_groups, group_body, 0)

        for c in (n_chunks - 2, n_chunks - 1):
            b = c % NBUF
            gather_desc(c, b).wait()
            scale_buf(b)
            write_desc(c, b).start()

        for c in (n_chunks - 3, n_chunks - 2, n_chunks - 1):
            write_desc(c, c % NBUF).wait()

    return k(x_flat, table)


def kernel(x, table):
    b, s = x.shape
    total_b = b * s
    x_flat = x.reshape(total_b).astype(jnp.int32)
    out = _embed(x_flat, table, total_b)
    return out.reshape(b, s, D_MODEL)


# speedup vs baseline: 1.0247x; 1.0247x over previous
import functools

import jax
import jax.numpy as jnp
from jax import lax
from jax.experimental import pallas as pl
from jax.experimental.pallas import tpu as pltpu
from jax.experimental.pallas import tpu_sc as plsc

D_MODEL = 1024
SCALE = 32.0

NUM_CORES = 2
NUM_SUBCORES = 16
LANES = 16
NW = NUM_CORES * NUM_SUBCORES

CHUNK = 16
NBUF = 4
NSLOT = 2


@functools.partial(jax.jit, static_argnames=("total_b",))
def _embed(x_flat, table, total_b):
    b_per_w = total_b // NW
    n_chunks = b_per_w // CHUNK
    n_groups = n_chunks // NBUF
    mesh = plsc.VectorSubcoreMesh(core_axis_name="c", subcore_axis_name="s")

    @functools.partial(
        pl.kernel,
        out_type=jax.ShapeDtypeStruct((total_b, D_MODEL), jnp.float32),
        mesh=mesh,
        scratch_types=[
            pltpu.VMEM((b_per_w,), jnp.int32),
            [pltpu.VMEM((CHUNK, D_MODEL), jnp.float32) for _ in range(NBUF)],
            pltpu.VMEM_SHARED((NUM_SUBCORES, NSLOT, CHUNK, D_MODEL), jnp.float32),
            [pltpu.SemaphoreType.DMA for _ in range(NBUF)],
            [pltpu.SemaphoreType.DMA for _ in range(NSLOT)],
        ],
    )
    def k(x_hbm, table_hbm, out_hbm, idx_v, rows, stage, gsems, wsems):
        sid = lax.axis_index("s")
        wid = sid * NUM_CORES + lax.axis_index("c")
        base = wid * b_per_w
        pltpu.sync_copy(x_hbm.at[pl.ds(base, b_per_w)], idx_v)

        def gather_desc(c, b):
            return pltpu.make_async_copy(
                table_hbm.at[idx_v.at[pl.ds(c * CHUNK, CHUNK)]], rows[b], gsems[b]
            )

        def write_desc(c, slot):
            return pltpu.make_async_copy(
                stage.at[sid, slot],
                out_hbm.at[pl.ds(base + c * CHUNK, CHUNK)],
                wsems[slot],
            )

        gather_desc(0, 0).start()
        gather_desc(1, 1).start()
        gather_desc(2, 2).start()

        def group_body(g, _):
            for b in range(NBUF):
                c = g * NBUF + b
                slot = b % NSLOT
                bp = (b + 3) % NBUF

                @pl.when(c + 3 < n_chunks)
                def _prefetch():
                    gather_desc(c + 3, bp).start()

                gather_desc(c, b).wait()

                @plsc.parallel_loop(0, CHUNK)
                def scale_row(r):
                    for j in range(D_MODEL // LANES):
                        v = rows[b][r, pl.ds(j * LANES, LANES)]
                        rows[b][r, pl.ds(j * LANES, LANES)] = v * SCALE

                @pl.when(c >= NSLOT)
                def _drain_write():
                    write_desc(c - NSLOT, slot).wait()

                pltpu.sync_copy(rows[b], stage.at[sid, slot])
                write_desc(c, slot).start()
            return 0

        lax.fori_loop(0, n_groups, group_body, 0)

        write_desc(n_chunks - 2, (n_chunks - 2) % NSLOT).wait()
        write_desc(n_chunks - 1, (n_chunks - 1) % NSLOT).wait()

    return k(x_flat, table)


def kernel(x, table):
    b, s = x.shape
    total_b = b * s
    x_flat = x.reshape(total_b).astype(jnp.int32)
    out = _embed(x_flat, table, total_b)
    return out.reshape(b, s, D_MODEL)
